# baseline (speedup 1.0000x reference)
import functools

import jax
import jax.numpy as jnp
import numpy as np
from jax import lax
from jax.experimental import pallas as pl
from jax.experimental.pallas import tpu as pltpu
from jax.experimental.pallas import tpu_sc as plsc

N = 1000000
D = 16
NSAMP = 25
B = 16384

_NC = 2
_NSUB = 16
_NW = _NC * _NSUB
_BPW = B // _NW
_CHUNK = 128
_NCHUNK = _BPW // _CHUNK

_LANE = 512
_NLANE = B // _LANE

_KS0 = np.uint32(0)
_KS1 = np.uint32(42)
_KS2 = np.uint32(0x1BD11BDA) ^ _KS0 ^ _KS1

_P_IN = [2.81022636e-08, 3.43273939e-07, -3.5233877e-06, -4.39150654e-06,
         0.00021858087, -0.00125372503, -0.00417768164, 0.246640727,
         1.50140941]
_P_OUT = [-0.000200214257, 0.000100950558, 0.00134934322, -0.00367342844,
          0.00573950773, -0.0076224613, 0.00943887047, 1.00167406,
          2.83297682]

_LO = np.float32(np.nextafter(np.float32(-1.0), np.float32(0.0)))
_SCALE = np.float32(np.float32(1.0) - _LO)
_SQRT2 = np.float32(np.sqrt(np.float32(2.0)))


def _sc_gather_body(idx_hbm, mu_hbm, sig_hbm, mu_out, sig_out,
                    idx_v, mu_v, sig_v, sem):
    wid = lax.axis_index("s") * _NC + lax.axis_index("c")
    base = wid * _BPW
    pltpu.sync_copy(idx_hbm.at[pl.ds(base, _BPW)], idx_v)
    copies = []
    for j in range(_NCHUNK):
        sl = pl.ds(j * _CHUNK, _CHUNK)
        copies.append(pltpu.async_copy(mu_hbm.at[idx_v.at[sl]], mu_v.at[sl], sem))
        copies.append(pltpu.async_copy(sig_hbm.at[idx_v.at[sl]], sig_v.at[sl], sem))
    for c in copies:
        c.wait()
    pltpu.sync_copy(mu_v, mu_out.at[pl.ds(base, _BPW)])
    pltpu.sync_copy(sig_v, sig_out.at[pl.ds(base, _BPW)])


def _sc_gather(indices, z_mu, z_log_sigma):
    mesh = plsc.VectorSubcoreMesh(core_axis_name="c", subcore_axis_name="s")
    run = functools.partial(
        pl.kernel,
        mesh=mesh,
        out_type=[
            jax.ShapeDtypeStruct((B, D), jnp.float32),
            jax.ShapeDtypeStruct((B, D), jnp.float32),
        ],
        compiler_params=pltpu.CompilerParams(use_tc_tiling_on_sc=False),
        scratch_types=[
            pltpu.VMEM((_BPW,), jnp.int32),
            pltpu.VMEM((_BPW, D), jnp.float32),
            pltpu.VMEM((_BPW, D), jnp.float32),
            pltpu.SemaphoreType.DMA,
        ],
    )(_sc_gather_body)
    return run(indices, z_mu, z_log_sigma)


def _rotl(x, r):
    return (x << r) | (x >> (32 - r))


def _threefry_rounds(x0, x1):
    for r in (13, 15, 26, 6):
        x0 = x0 + x1
        x1 = _rotl(x1, r)
        x1 = x1 ^ x0
    x0 = x0 + _KS1
    x1 = x1 + (_KS2 + np.uint32(1))
    for r in (17, 29, 16, 24):
        x0 = x0 + x1
        x1 = _rotl(x1, r)
        x1 = x1 ^ x0
    x0 = x0 + _KS2
    x1 = x1 + (_KS0 + np.uint32(2))
    for r in (13, 15, 26, 6):
        x0 = x0 + x1
        x1 = _rotl(x1, r)
        x1 = x1 ^ x0
    x0 = x0 + _KS0
    x1 = x1 + (_KS1 + np.uint32(3))
    for r in (17, 29, 16, 24):
        x0 = x0 + x1
        x1 = _rotl(x1, r)
        x1 = x1 ^ x0
    x0 = x0 + _KS1
    x1 = x1 + (_KS2 + np.uint32(4))
    for r in (13, 15, 26, 6):
        x0 = x0 + x1
        x1 = _rotl(x1, r)
        x1 = x1 ^ x0
    x0 = x0 + _KS2
    x1 = x1 + (_KS0 + np.uint32(5))
    return x0, x1


def _tc_body(mu_ref, sig_ref, out_ref, std_ref):
    s = pl.program_id(0)

    @pl.when(s == 0)
    def _():
        sig = sig_ref[...]
        sp = jnp.log1p(jnp.exp(-jnp.abs(sig))) + jnp.maximum(sig, 0.0)
        std_ref[...] = sp * _SQRT2

    base = (s * (B * D)).astype(jnp.uint32)

    def chunk(c, _):
        d_io = lax.broadcasted_iota(jnp.uint32, (D, _LANE), 0)
        b_io = lax.broadcasted_iota(jnp.uint32, (D, _LANE), 1)
        i = base + (b_io + np.uint32(_LANE) * c.astype(jnp.uint32)) * np.uint32(D) + d_io
        x1 = i + _KS1
        x0 = jnp.broadcast_to(_KS0, (D, _LANE)).astype(jnp.uint32)
        x0, x1 = _threefry_rounds(x0, x1)
        bits = x0 ^ x1
        u01 = lax.bitcast_convert_type((bits >> 9) | np.uint32(0x3F800000),
                                       jnp.float32) - np.float32(1.0)
        u = jnp.maximum(_LO, u01 * _SCALE + _LO)
        w = -jnp.log1p(-u * u)
        in_range = w < np.float32(5.0)
        t = jnp.where(in_range, w - np.float32(2.5),
                      jnp.sqrt(w) - np.float32(3.0))
        p = jnp.where(in_range, np.float32(_P_IN[0]), np.float32(_P_OUT[0]))
        for a, bco in zip(_P_IN[1:], _P_OUT[1:]):
            csel = jnp.where(in_range, np.float32(a), np.float32(bco))
            p = p * t + csel
        pu = p * u
        sl = pl.ds(c * _LANE, _LANE)
        out_ref[0, :, sl] = mu_ref[:, sl] + std_ref[:, sl] * pu
        return 0

    lax.fori_loop(0, _NLANE, chunk, 0)


def _tc_fused(mu_t, sig_t):
    return pl.pallas_call(
        _tc_body,
        grid=(NSAMP,),
        in_specs=[
            pl.BlockSpec((D, B), lambda i: (0, 0)),
            pl.BlockSpec((D, B), lambda i: (0, 0)),
        ],
        out_specs=pl.BlockSpec((1, D, B), lambda i: (i, 0, 0)),
        out_shape=jax.ShapeDtypeStruct((NSAMP, D, B), jnp.float32),
        scratch_shapes=[pltpu.VMEM((D, B), jnp.float32)],
    )(mu_t, sig_t)


def kernel(indices, z_mu, z_log_sigma):
    mu_g, sig_g = _sc_gather(indices, z_mu, z_log_sigma)
    out3 = _tc_fused(mu_g.T, sig_g.T)
    return out3.transpose(0, 2, 1)


# speedup vs baseline: 1.0167x; 1.0167x over previous
import functools

import jax
import jax.numpy as jnp
import numpy as np
from jax import lax
from jax.experimental import pallas as pl
from jax.experimental.pallas import tpu as pltpu
from jax.experimental.pallas import tpu_sc as plsc

N = 1000000
D = 16
NSAMP = 25
B = 16384

_NC = 2
_NSUB = 16
_NW = _NC * _NSUB
_BPW = B // _NW
_CHUNK = 128
_NCHUNK = _BPW // _CHUNK

_LANE = 2048
_NLANE = B // _LANE

_KS0 = np.uint32(0)
_KS1 = np.uint32(42)
_KS2 = np.uint32(0x1BD11BDA) ^ _KS0 ^ _KS1

_P_IN = [2.81022636e-08, 3.43273939e-07, -3.5233877e-06, -4.39150654e-06,
         0.00021858087, -0.00125372503, -0.00417768164, 0.246640727,
         1.50140941]
_P_OUT = [-0.000200214257, 0.000100950558, 0.00134934322, -0.00367342844,
          0.00573950773, -0.0076224613, 0.00943887047, 1.00167406,
          2.83297682]

_LO = np.float32(np.nextafter(np.float32(-1.0), np.float32(0.0)))
_SCALE = np.float32(np.float32(1.0) - _LO)
_SQRT2 = np.float32(np.sqrt(np.float32(2.0)))


_NBIG = N // 8


def _sc_gather_body(idx_hbm, mu_hbm, sig_hbm, mu_out, sig_out,
                    idx_v, bidx_v, big_v, st_v, sem):
    wid = lax.axis_index("s") * _NC + lax.axis_index("c")
    base = wid * _BPW
    pltpu.sync_copy(idx_hbm.at[pl.ds(base, _BPW)], idx_v)
    for k in range(_BPW // 16):
        sl = pl.ds(k * 16, 16)
        bidx_v[sl] = idx_v[sl] >> 3

    def one_table(tab_hbm, tab_out):
        copies = []
        for j in range(_NCHUNK):
            sl = pl.ds(j * _CHUNK, _CHUNK)
            copies.append(pltpu.async_copy(
                tab_hbm.at[bidx_v.at[sl]], big_v.at[sl], sem))
        for c in copies:
            c.wait()

        def extract(g, _):
            sl = pl.ds(g * 16, 16)
            rows = lax.iota(jnp.int32, 16) + g * 16
            sub16 = (idx_v[sl] & 7) << 4
            for c in range(D):
                v = plsc.load_gather(big_v, [rows, sub16 + c])
                plsc.store_scatter(st_v, [rows, jnp.full((16,), c, jnp.int32)], v)
            return 0

        lax.fori_loop(0, _BPW // 16, extract, 0)
        pltpu.sync_copy(st_v, tab_out.at[pl.ds(base, _BPW)])

    one_table(mu_hbm, mu_out)
    one_table(sig_hbm, sig_out)


def _sc_gather(indices, z_mu, z_log_sigma):
    mesh = plsc.VectorSubcoreMesh(core_axis_name="c", subcore_axis_name="s")
    run = functools.partial(
        pl.kernel,
        mesh=mesh,
        out_type=[
            jax.ShapeDtypeStruct((B, D), jnp.float32),
            jax.ShapeDtypeStruct((B, D), jnp.float32),
        ],
        compiler_params=pltpu.CompilerParams(use_tc_tiling_on_sc=False,
                                             needs_layout_passes=False),
        scratch_types=[
            pltpu.VMEM((_BPW,), jnp.int32),
            pltpu.VMEM((_BPW,), jnp.int32),
            pltpu.VMEM((_BPW, 128), jnp.float32),
            pltpu.VMEM((_BPW, D), jnp.float32),
            pltpu.SemaphoreType.DMA,
        ],
    )(_sc_gather_body)
    mu_rs = z_mu.reshape(_NBIG, 128)
    sig_rs = z_log_sigma.reshape(_NBIG, 128)
    return run(indices, mu_rs, sig_rs)


def _rotl(x, r):
    return (x << r) | (x >> (32 - r))


def _threefry_rounds(x0, x1):
    for r in (13, 15, 26, 6):
        x0 = x0 + x1
        x1 = _rotl(x1, r)
        x1 = x1 ^ x0
    x0 = x0 + _KS1
    x1 = x1 + (_KS2 + np.uint32(1))
    for r in (17, 29, 16, 24):
        x0 = x0 + x1
        x1 = _rotl(x1, r)
        x1 = x1 ^ x0
    x0 = x0 + _KS2
    x1 = x1 + (_KS0 + np.uint32(2))
    for r in (13, 15, 26, 6):
        x0 = x0 + x1
        x1 = _rotl(x1, r)
        x1 = x1 ^ x0
    x0 = x0 + _KS0
    x1 = x1 + (_KS1 + np.uint32(3))
    for r in (17, 29, 16, 24):
        x0 = x0 + x1
        x1 = _rotl(x1, r)
        x1 = x1 ^ x0
    x0 = x0 + _KS1
    x1 = x1 + (_KS2 + np.uint32(4))
    for r in (13, 15, 26, 6):
        x0 = x0 + x1
        x1 = _rotl(x1, r)
        x1 = x1 ^ x0
    x0 = x0 + _KS2
    x1 = x1 + (_KS0 + np.uint32(5))
    return x0, x1


def _tc_body(mu_ref, sig_ref, out_ref, std_ref):
    s = pl.program_id(0)

    @pl.when(s == 0)
    def _():
        sig = sig_ref[...]
        sp = jnp.log1p(jnp.exp(-jnp.abs(sig))) + jnp.maximum(sig, 0.0)
        std_ref[...] = sp * _SQRT2

    base = (s * (B * D)).astype(jnp.uint32)
    d_io = lax.broadcasted_iota(jnp.uint32, (D, _LANE), 0)
    b_io = lax.broadcasted_iota(jnp.uint32, (D, _LANE), 1)
    r_io = b_io * np.uint32(D) + d_io

    def chunk(c, _):
        i = (base + np.uint32(_LANE * D) * c.astype(jnp.uint32)) + r_io
        x1 = i + _KS1
        x0 = jnp.broadcast_to(_KS0, (D, _LANE)).astype(jnp.uint32)
        x0, x1 = _threefry_rounds(x0, x1)
        bits = x0 ^ x1
        u01 = lax.bitcast_convert_type((bits >> 9) | np.uint32(0x3F800000),
                                       jnp.float32) - np.float32(1.0)
        u = u01 * _SCALE + _LO
        w = -jnp.log1p(-u * u)
        in_range = w < np.float32(5.0)
        t = jnp.where(in_range, w - np.float32(2.5),
                      jnp.sqrt(w) - np.float32(3.0))
        p = jnp.where(in_range, np.float32(_P_IN[0]), np.float32(_P_OUT[0]))
        for a, bco in zip(_P_IN[1:], _P_OUT[1:]):
            csel = jnp.where(in_range, np.float32(a), np.float32(bco))
            p = p * t + csel
        pu = p * u
        sl = pl.ds(c * _LANE, _LANE)
        out_ref[0, :, sl] = mu_ref[:, sl] + std_ref[:, sl] * pu
        return 0

    lax.fori_loop(0, _NLANE, chunk, 0)


def _tc_fused(mu_t, sig_t):
    return pl.pallas_call(
        _tc_body,
        grid=(NSAMP,),
        in_specs=[
            pl.BlockSpec((D, B), lambda i: (0, 0)),
            pl.BlockSpec((D, B), lambda i: (0, 0)),
        ],
        out_specs=pl.BlockSpec((1, D, B), lambda i: (i, 0, 0)),
        out_shape=jax.ShapeDtypeStruct((NSAMP, D, B), jnp.float32),
        scratch_shapes=[pltpu.VMEM((D, B), jnp.float32)],
    )(mu_t, sig_t)


def kernel(indices, z_mu, z_log_sigma):
    mu_g, sig_g = _sc_gather(indices, z_mu, z_log_sigma)
    out3 = _tc_fused(mu_g.T, sig_g.T)
    return out3.transpose(0, 2, 1)
